# baseline (speedup 1.0000x reference)
import functools

import jax
import jax.numpy as jnp
from jax import lax
from jax.experimental import pallas as pl
from jax.experimental.pallas import tpu as pltpu
from jax.experimental.pallas import tpu_sc as plsc

B = 16384
NCAT = 26
VOCAB = 100000
VOCAB_PAD = 100096
D = 32
NCONT = 13
NBINS = 8
NTOK = NCAT + NCONT

NC, NS = 2, 16
NW = NC * NS
ROWS = B * NCAT
ROWS_PER_W = ROWS // NW
CHUNK = 128
STEPS = ROWS_PER_W // CHUNK

VB = 5888
NVB = VOCAB_PAD // VB
BB = 512


def _detile_kernel(t_ref, o_ref):
    o_ref[...] = jnp.swapaxes(t_ref[...], 1, 2)


def _detile(t2):
    return pl.pallas_call(
        _detile_kernel,
        grid=(NCAT, NVB),
        in_specs=[pl.BlockSpec((1, D, VB), lambda f, v: (f, 0, v))],
        out_specs=pl.BlockSpec((1, VB, D), lambda f, v: (f, v, 0)),
        out_shape=jax.ShapeDtypeStruct((NCAT, VOCAB_PAD, D), jnp.float32),
    )(t2)


def _gather_body(table_h, idx_h, out_h, idx_v, buf_v, gsem, wsem):
    wid = lax.axis_index("s") * NC + lax.axis_index("c")
    base = wid * ROWS_PER_W
    pltpu.sync_copy(idx_h.at[wid], idx_v)

    def step(j, carry):
        pltpu.async_copy(table_h.at[idx_v.at[j]], buf_v, gsem).wait()
        pltpu.async_copy(buf_v, out_h.at[pl.ds(base + j * CHUNK, CHUNK)],
                         wsem).wait()
        return carry

    lax.fori_loop(0, STEPS, step, 0)


@functools.cache
def _sc_gather():
    return pl.kernel(
        _gather_body,
        out_type=jax.ShapeDtypeStruct((ROWS, D), jnp.float32),
        mesh=plsc.VectorSubcoreMesh(core_axis_name="c", subcore_axis_name="s",
                                    num_cores=NC, num_subcores=NS),
        compiler_params=pltpu.CompilerParams(use_tc_tiling_on_sc=False),
        scratch_types=[
            pltpu.VMEM((STEPS, CHUNK), jnp.int32),
            pltpu.VMEM((CHUNK, D), jnp.float32),
            pltpu.SemaphoreType.DMA,
            pltpu.SemaphoreType.DMA,
        ],
    )


def _pack_kernel(g_ref, xrep_ref, bb_ref, w_ref, b_ref, o_ref):
    o_ref[0:NCAT] = jnp.swapaxes(g_ref[...], 1, 2)
    enc = jnp.maximum(1.0 - jnp.abs(xrep_ref[...] - bb_ref[...]), 0.0)
    cont = lax.dot_general(w_ref[...], enc, (((0,), (0,)), ((), ())),
                           preferred_element_type=jnp.float32)
    cont = cont + b_ref[...]
    o_ref[NCAT:NTOK] = cont.reshape(NCONT, D, BB)


def _pack(g3, x_rep_t, bb_col, W, b_col):
    nf = NCONT * NBINS
    return pl.pallas_call(
        _pack_kernel,
        grid=(B // BB,),
        in_specs=[
            pl.BlockSpec((NCAT, BB, D), lambda i: (0, i, 0)),
            pl.BlockSpec((nf, BB), lambda i: (0, i)),
            pl.BlockSpec((nf, 1), lambda i: (0, 0)),
            pl.BlockSpec((nf, NCONT * D), lambda i: (0, 0)),
            pl.BlockSpec((NCONT * D, 1), lambda i: (0, 0)),
        ],
        out_specs=pl.BlockSpec((NTOK, D, BB), lambda i: (0, 0, i)),
        out_shape=jax.ShapeDtypeStruct((NTOK, D, B), jnp.float32),
    )(g3, x_rep_t, bb_col, W, b_col)


def kernel(x_cat, x_cont, tables, bin_boundaries, W, b):
    t2 = jnp.transpose(tables, (0, 2, 1))
    table_rows = _detile(t2).reshape(NCAT * VOCAB_PAD, D)

    idx_t = (x_cat.T.astype(jnp.int32)
             + (jnp.arange(NCAT, dtype=jnp.int32) * VOCAB_PAD)[:, None])
    idx3 = idx_t.reshape(NW, STEPS, CHUNK)
    g_flat = _sc_gather()(table_rows, idx3)

    x_rep_t = jnp.repeat(x_cont.T, NBINS, axis=0)
    bb_col = bin_boundaries.reshape(NCONT * NBINS, 1)
    out_phys = _pack(g_flat.reshape(NCAT, B, D), x_rep_t, bb_col,
                     W, b.reshape(NCONT * D, 1))
    return jnp.transpose(out_phys, (2, 0, 1))


# speedup vs baseline: 1.9915x; 1.9915x over previous
import functools

import jax
import jax.numpy as jnp
from jax import lax
from jax.experimental import pallas as pl
from jax.experimental.pallas import tpu as pltpu
from jax.experimental.pallas import tpu_sc as plsc

B = 16384
NCAT = 26
VOCAB = 100000
VOCAB_PAD = 100096
D = 32
NCONT = 13
NBINS = 8
NTOK = NCAT + NCONT

NC, NS = 2, 16
NW = NC * NS
ROWS = B * NCAT
ROWS_PER_W = ROWS // NW
CHUNK = 128
STEPS = ROWS_PER_W // CHUNK

VB = 5888
VB4 = VB // 4
NVB = VOCAB_PAD // VB
RPF = VOCAB_PAD // 4
BB = 512
BQ = BB // 4


def _detile_kernel(t_ref, o_ref):
    x = t_ref[...]
    parts = [jnp.swapaxes(x[:, :, k * VB4:(k + 1) * VB4], 1, 2)
             for k in range(4)]
    o_ref[...] = jnp.concatenate(parts, axis=2)


def _detile(t2):
    return pl.pallas_call(
        _detile_kernel,
        grid=(NCAT, NVB),
        in_specs=[pl.BlockSpec((1, D, VB), lambda f, v: (f, 0, v))],
        out_specs=pl.BlockSpec((1, VB4, 4 * D), lambda f, v: (f, v, 0)),
        out_shape=jax.ShapeDtypeStruct((NCAT, RPF, 4 * D), jnp.float32),
    )(t2)


def _gather_body(table_h, idx_h, out_h, idx_v, buf_v, gsem, wsem):
    wid = lax.axis_index("s") * NC + lax.axis_index("c")
    base = wid * ROWS_PER_W
    pltpu.sync_copy(idx_h.at[wid], idx_v)

    def step(j, carry):
        pltpu.async_copy(table_h.at[idx_v.at[j]], buf_v, gsem).wait()
        pltpu.async_copy(buf_v, out_h.at[pl.ds(base + j * CHUNK, CHUNK)],
                         wsem).wait()
        return carry

    lax.fori_loop(0, STEPS, step, 0)


@functools.cache
def _sc_gather():
    return pl.kernel(
        _gather_body,
        out_type=jax.ShapeDtypeStruct((ROWS, D), jnp.float32),
        mesh=plsc.VectorSubcoreMesh(core_axis_name="c", subcore_axis_name="s",
                                    num_cores=NC, num_subcores=NS),
        compiler_params=pltpu.CompilerParams(use_tc_tiling_on_sc=False),
        scratch_types=[
            pltpu.VMEM((STEPS, CHUNK), jnp.int32),
            pltpu.VMEM((CHUNK, D), jnp.float32),
            pltpu.SemaphoreType.DMA,
            pltpu.SemaphoreType.DMA,
        ],
    )


def _pack_kernel(g_ref, xrep_ref, bb_ref, w_ref, b_ref, o_ref):
    y = jnp.swapaxes(g_ref[:, 0], 1, 2)
    for k in range(4):
        o_ref[0:NCAT, :, k * BQ:(k + 1) * BQ] = (
            y[:, k * D:(k + 1) * D, :])
    enc = jnp.maximum(1.0 - jnp.abs(xrep_ref[...] - bb_ref[...]), 0.0)
    cont = lax.dot_general(w_ref[...], enc, (((0,), (0,)), ((), ())),
                           preferred_element_type=jnp.float32)
    cont = cont + b_ref[...]
    o_ref[NCAT:NTOK] = cont.reshape(NCONT, D, BB)


def _pack(g4, x_rep_t, bb_col, W, b_col):
    nf = NCONT * NBINS
    return pl.pallas_call(
        _pack_kernel,
        grid=(B // BB,),
        in_specs=[
            pl.BlockSpec((NCAT, 1, BQ, 128), lambda i: (0, i, 0, 0)),
            pl.BlockSpec((nf, BB), lambda i: (0, i)),
            pl.BlockSpec((nf, 1), lambda i: (0, 0)),
            pl.BlockSpec((nf, NCONT * D), lambda i: (0, 0)),
            pl.BlockSpec((NCONT * D, 1), lambda i: (0, 0)),
        ],
        out_specs=pl.BlockSpec((NTOK, D, BB), lambda i: (0, 0, i)),
        out_shape=jax.ShapeDtypeStruct((NTOK, D, B), jnp.float32),
    )(g4, x_rep_t, bb_col, W, b_col)


def kernel(x_cat, x_cont, tables, bin_boundaries, W, b):
    t2 = jnp.transpose(tables, (0, 2, 1))
    table_rows = _detile(t2).reshape(NCAT * VOCAB_PAD, D)

    v = x_cat.T.astype(jnp.int32)
    vi, w = v // VB, v % VB
    k, u = w // VB4, w % VB4
    r32 = ((jnp.arange(NCAT, dtype=jnp.int32) * RPF)[:, None]
           + vi * VB4 + u) * 4 + k
    perm = (r32.reshape(NCAT, B // BB, 4, BQ)
            .transpose(0, 1, 3, 2))
    idx3 = perm.reshape(NW, STEPS, CHUNK)
    g_flat = _sc_gather()(table_rows, idx3)

    x_rep_t = jnp.repeat(x_cont.T, NBINS, axis=0)
    bb_col = bin_boundaries.reshape(NCONT * NBINS, 1)
    out_phys = _pack(g_flat.reshape(NCAT, B // BB, BQ, 128), x_rep_t, bb_col,
                     W, b.reshape(NCONT * D, 1))
    return jnp.transpose(out_phys, (2, 0, 1))
